# baseline (speedup 1.0000x reference)
import functools

import jax
import jax.numpy as jnp
from jax import lax
from jax.experimental import pallas as pl
from jax.experimental.pallas import tpu as pltpu
from jax.experimental.pallas import tpu_sc as plsc

_B = 4096
_S = 200
_D = 64
_SCALE = 8.0

_NC = 2
_NS = 16
_NW = _NC * _NS
_R = _B * _S
_RPW = _R // _NW
_C = 128
_CPW = _RPW // _C

_mesh = plsc.VectorSubcoreMesh(core_axis_name="c", subcore_axis_name="s")


@functools.partial(
    pl.kernel,
    out_type=jax.ShapeDtypeStruct((_R, _D), jnp.float32),
    mesh=_mesh,
    compiler_params=pltpu.CompilerParams(
        needs_layout_passes=False, use_tc_tiling_on_sc=False),
    scratch_types=[
        pltpu.VMEM((_C,), jnp.int32),
        pltpu.VMEM((_C, _D), jnp.float32),
        pltpu.VMEM((_S, _D), jnp.float32),
        pltpu.SemaphoreType.DMA,
    ],
)
def _embed(ids_hbm, tok_hbm, pos_hbm, out_hbm, idx_v, rows_v, pos_v, sem):
    wid = lax.axis_index("s") * _NC + lax.axis_index("c")
    base = wid * _RPW
    pltpu.sync_copy(pos_hbm, pos_v)

    def chunk_body(c, carry):
        row0 = base + c * _C
        pltpu.sync_copy(ids_hbm.at[pl.ds(row0, _C)], idx_v)
        pltpu.async_copy(tok_hbm.at[idx_v], rows_v, sem).wait()
        s0 = lax.rem(row0, _S)

        @plsc.parallel_loop(0, _C)
        def _row(i):
            s = s0 + i
            s = jnp.where(s >= _S, s - _S, s)
            tok = plsc.load_gather(idx_v, [jnp.full((16,), i, jnp.int32)])
            m8 = jnp.where(tok != 0, _SCALE, 0.0)
            m1 = jnp.where(tok != 0, 1.0, 0.0)
            for j in range(_D // 16):
                sl = pl.ds(j * 16, 16)
                rows_v[i, sl] = rows_v[i, sl] * m8 + pos_v[s, sl] * m1

        pltpu.sync_copy(rows_v, out_hbm.at[pl.ds(row0, _C)])
        return carry

    lax.fori_loop(0, _CPW, chunk_body, 0)


def kernel(inputs, token_table, pos_table):
    flat_ids = inputs.reshape(_R)
    out = _embed(flat_ids, token_table, pos_table)
    return out.reshape(_B, _S, _D)
